# baseline (speedup 1.0000x reference)
import functools

import jax
import jax.numpy as jnp
from jax import lax
from jax.experimental import pallas as pl
from jax.experimental.pallas import tpu as pltpu
from jax.experimental.pallas import tpu_sc as plsc

B = 4
S = 4096
N = 2048
D = 768

NW = 32
SPW = B * N // NW
CS = 32
NCHUNK = SPW // CS
L = 16

_mesh = plsc.VectorSubcoreMesh(core_axis_name="c", subcore_axis_name="s")

_GDN = lax.GatherDimensionNumbers(
    offset_dims=(), collapsed_slice_dims=(0,), start_index_map=(0,)
)


def _vperm(x, perm):
    return lax.gather(
        x, perm[:, None], _GDN, (1,),
        mode=lax.GatherScatterMode.PROMISE_IN_BOUNDS,
    )


@functools.partial(
    pl.kernel,
    mesh=_mesh,
    out_type=jax.ShapeDtypeStruct((B, N, 2 * D), jnp.float32),
    scratch_types=[
        pltpu.VMEM((2 * SPW,), jnp.int32),
        pltpu.VMEM((SPW,), jnp.int32),
        pltpu.VMEM((SPW,), jnp.int32),
        pltpu.VMEM((CS, D), jnp.float32),
        pltpu.VMEM((CS, D), jnp.float32),
        pltpu.VMEM((CS, D), jnp.float32),
        pltpu.VMEM((CS, D), jnp.float32),
        pltpu.SemaphoreType.DMA,
        pltpu.SemaphoreType.DMA,
        pltpu.SemaphoreType.DMA,
        pltpu.SemaphoreType.DMA,
        pltpu.SemaphoreType.DMA,
        pltpu.SemaphoreType.DMA,
        pltpu.SemaphoreType.DMA,
        pltpu.SemaphoreType.DMA,
    ],
)
def _sc_gather(
    table_hbm, idx_hbm, out_hbm,
    idx_v, idx_s, idx_e, bs0, bs1, be0, be1,
    sem_s0, sem_s1, sem_e0, sem_e1, wsem_s0, wsem_s1, wsem_e0, wsem_e1,
):
    wid = lax.axis_index("s") * 2 + lax.axis_index("c")
    sbase = wid * SPW
    b = sbase // N
    nb = sbase - b * N
    pltpu.sync_copy(idx_hbm.at[pl.ds(2 * sbase, 2 * SPW)], idx_v)
    boff = b * S
    lane = lax.iota(jnp.int32, L)
    perm_even = (lane % 8) * 2
    perm_odd = perm_even + 1
    in_low = lane < 8
    for i in range(SPW // L):
        v0 = idx_v[pl.ds(2 * i * L, L)]
        v1 = idx_v[pl.ds(2 * i * L + L, L)]
        idx_s[pl.ds(i * L, L)] = (
            jnp.where(in_low, _vperm(v0, perm_even), _vperm(v1, perm_even)) + boff
        )
        idx_e[pl.ds(i * L, L)] = (
            jnp.where(in_low, _vperm(v0, perm_odd), _vperm(v1, perm_odd)) + boff
        )

    bufs_s = (bs0, bs1)
    bufs_e = (be0, be1)
    sems_s = (sem_s0, sem_s1)
    sems_e = (sem_e0, sem_e1)
    wsems_s = (wsem_s0, wsem_s1)
    wsems_e = (wsem_e0, wsem_e1)

    def start_gathers(ci):
        k = ci % 2
        hs = pltpu.async_copy(
            table_hbm.at[idx_s.at[pl.ds(ci * CS, CS)]], bufs_s[k], sems_s[k]
        )
        he = pltpu.async_copy(
            table_hbm.at[idx_e.at[pl.ds(ci * CS, CS)]], bufs_e[k], sems_e[k]
        )
        return hs, he

    writes = {}
    pending = start_gathers(0)
    for ci in range(NCHUNK):
        cur_s, cur_e = pending
        if ci + 1 < NCHUNK:
            if ci - 1 in writes:
                for h in writes.pop(ci - 1):
                    h.wait()
            pending = start_gathers(ci + 1)
        cur_s.wait()
        cur_e.wait()
        k = ci % 2
        row0 = nb + ci * CS
        writes[ci] = (
            pltpu.async_copy(
                bufs_s[k], out_hbm.at[b, pl.ds(row0, CS), pl.ds(0, D)], wsems_s[k]
            ),
            pltpu.async_copy(
                bufs_e[k], out_hbm.at[b, pl.ds(row0, CS), pl.ds(D, D)], wsems_e[k]
            ),
        )
    for ci in sorted(writes):
        for h in writes[ci]:
            h.wait()


def kernel(sequence_tensor, span_indices):
    table = sequence_tensor.reshape(B * S, D)
    idx = span_indices.astype(jnp.int32).reshape(2 * B * N)
    return _sc_gather(table, idx)


# speedup vs baseline: 1.0473x; 1.0473x over previous
import functools

import jax
import jax.numpy as jnp
from jax import lax
from jax.experimental import pallas as pl
from jax.experimental.pallas import tpu as pltpu
from jax.experimental.pallas import tpu_sc as plsc

B = 4
S = 4096
N = 2048
D = 768

NW = 32
SPW = B * N // NW
CS = 32
NCHUNK = SPW // CS
L = 16

_mesh = plsc.VectorSubcoreMesh(core_axis_name="c", subcore_axis_name="s")


@functools.partial(
    pl.kernel,
    mesh=_mesh,
    out_type=jax.ShapeDtypeStruct((B, N, 2 * D), jnp.float32),
    scratch_types=[
        pltpu.VMEM((SPW,), jnp.int32),
        pltpu.VMEM((SPW,), jnp.int32),
        pltpu.VMEM((CS, 2 * D), jnp.float32),
        pltpu.VMEM((CS, 2 * D), jnp.float32),
        pltpu.SemaphoreType.DMA,
        pltpu.SemaphoreType.DMA,
        pltpu.SemaphoreType.DMA,
        pltpu.SemaphoreType.DMA,
    ],
)
def _sc_gather(
    table_hbm, sidx_hbm, eidx_hbm, out_hbm,
    idx_s, idx_e, buf0, buf1, gsem0, gsem1, wsem0, wsem1,
):
    wid = lax.axis_index("s") * 2 + lax.axis_index("c")
    sbase = wid * SPW
    b = sbase // N
    nb = sbase - b * N
    pltpu.sync_copy(sidx_hbm.at[pl.ds(sbase, SPW)], idx_s)
    pltpu.sync_copy(eidx_hbm.at[pl.ds(sbase, SPW)], idx_e)
    boff = b * S
    for i in range(SPW // L):
        idx_s[pl.ds(i * L, L)] = idx_s[pl.ds(i * L, L)] + boff
        idx_e[pl.ds(i * L, L)] = idx_e[pl.ds(i * L, L)] + boff

    bufs = (buf0, buf1)
    gsems = (gsem0, gsem1)
    wsems = (wsem0, wsem1)

    def start_gathers(ci):
        k = ci % 2
        hs = pltpu.async_copy(
            table_hbm.at[idx_s.at[pl.ds(ci * CS, CS)]],
            bufs[k].at[:, pl.ds(0, D)],
            gsems[k],
        )
        he = pltpu.async_copy(
            table_hbm.at[idx_e.at[pl.ds(ci * CS, CS)]],
            bufs[k].at[:, pl.ds(D, D)],
            gsems[k],
        )
        return hs, he

    writes = {}
    pending = start_gathers(0)
    for ci in range(NCHUNK):
        cur_s, cur_e = pending
        if ci + 1 < NCHUNK:
            if ci - 1 in writes:
                writes.pop(ci - 1).wait()
            pending = start_gathers(ci + 1)
        cur_s.wait()
        cur_e.wait()
        k = ci % 2
        writes[ci] = pltpu.async_copy(
            bufs[k], out_hbm.at[b, pl.ds(nb + ci * CS, CS), :], wsems[k]
        )
    for ci in sorted(writes):
        writes[ci].wait()


def kernel(sequence_tensor, span_indices):
    table = sequence_tensor.reshape(B * S, D)
    si = span_indices.astype(jnp.int32)
    starts = si[..., 0].reshape(B * N)
    ends = si[..., 1].reshape(B * N)
    return _sc_gather(table, starts, ends)


# speedup vs baseline: 1.0533x; 1.0058x over previous
import functools

import jax
import jax.numpy as jnp
from jax import lax
from jax.experimental import pallas as pl
from jax.experimental.pallas import tpu as pltpu
from jax.experimental.pallas import tpu_sc as plsc

B = 4
S = 4096
N = 2048
D = 768

NW = 32
SPW = B * N // NW
CS = 16
NCHUNK = SPW // CS
NBUF = 4
L = 16

_mesh = plsc.VectorSubcoreMesh(core_axis_name="c", subcore_axis_name="s")


@functools.partial(
    pl.kernel,
    mesh=_mesh,
    out_type=jax.ShapeDtypeStruct((B, N, 2 * D), jnp.float32),
    scratch_types=[
        pltpu.VMEM((SPW,), jnp.int32),
        pltpu.VMEM((SPW,), jnp.int32),
        pltpu.VMEM((CS, 2 * D), jnp.float32),
        pltpu.VMEM((CS, 2 * D), jnp.float32),
        pltpu.VMEM((CS, 2 * D), jnp.float32),
        pltpu.VMEM((CS, 2 * D), jnp.float32),
    ]
    + [pltpu.SemaphoreType.DMA] * 8,
)
def _sc_gather(
    table_hbm, sidx_hbm, eidx_hbm, out_hbm,
    idx_s, idx_e, buf0, buf1, buf2, buf3, *sems,
):
    wid = lax.axis_index("s") * 2 + lax.axis_index("c")
    sbase = wid * SPW
    b = sbase // N
    nb = sbase - b * N
    pltpu.sync_copy(sidx_hbm.at[pl.ds(sbase, SPW)], idx_s)
    pltpu.sync_copy(eidx_hbm.at[pl.ds(sbase, SPW)], idx_e)
    boff = b * S
    for i in range(SPW // L):
        idx_s[pl.ds(i * L, L)] = idx_s[pl.ds(i * L, L)] + boff
        idx_e[pl.ds(i * L, L)] = idx_e[pl.ds(i * L, L)] + boff

    bufs = (buf0, buf1, buf2, buf3)
    gsems = sems[:4]
    wsems = sems[4:]

    def start_gathers(ci):
        k = ci % NBUF
        hs = pltpu.async_copy(
            table_hbm.at[idx_s.at[pl.ds(ci * CS, CS)]],
            bufs[k].at[:, pl.ds(0, D)],
            gsems[k],
        )
        he = pltpu.async_copy(
            table_hbm.at[idx_e.at[pl.ds(ci * CS, CS)]],
            bufs[k].at[:, pl.ds(D, D)],
            gsems[k],
        )
        return hs, he

    writes = {}
    pending = {}
    for j in range(NBUF - 1):
        pending[j] = start_gathers(j)
    for ci in range(NCHUNK):
        nxt = ci + NBUF - 1
        if nxt < NCHUNK:
            if nxt - NBUF in writes:
                writes.pop(nxt - NBUF).wait()
            pending[nxt] = start_gathers(nxt)
        cur_s, cur_e = pending.pop(ci)
        cur_s.wait()
        cur_e.wait()
        k = ci % NBUF
        writes[ci] = pltpu.async_copy(
            bufs[k], out_hbm.at[b, pl.ds(nb + ci * CS, CS), :], wsems[k]
        )
    for ci in sorted(writes):
        writes[ci].wait()


def kernel(sequence_tensor, span_indices):
    table = sequence_tensor.reshape(B * S, D)
    si = span_indices.astype(jnp.int32)
    starts = si[..., 0].reshape(B * N)
    ends = si[..., 1].reshape(B * N)
    return _sc_gather(table, starts, ends)
